# baseline (speedup 1.0000x reference)
import jax
import jax.numpy as jnp
from jax.experimental import pallas as pl
from jax.experimental.pallas import tpu as pltpu

_PI = 3.141592653589793
_B = 32
_D = 16

_S = (0.9999994487625018, -0.1666652052026121, 0.008332213456463705,
      -0.0001980409467322523, 2.6949810250816164e-06,
      -2.0183356264080743e-08)
_C = (0.9999988662725678, -0.49999236445676765, 0.04165820791662088,
      -0.0013854161244634651, 2.4147561505205584e-05,
      -2.1742084399534178e-07)


def _poly_sin_cos(p):
    x2 = p * p
    sp = _S[5]
    for c in _S[4::-1]:
        sp = sp * x2 + c
    sp = sp * p
    cp = _C[5]
    for c in _C[4::-1]:
        cp = cp * x2 + c
    return sp, cp


def _prep_body(triples_ref, entT_ref, relT_ref, ent_hbm, rel_hbm,
               sinA_ref, cosA_ref, scale_ref, heads, rels, sem):
    for b in range(_B):
        hi = triples_ref[b, 0]
        ri = triples_ref[b, 1]
        pltpu.make_async_copy(ent_hbm.at[pl.ds(hi, 1), :],
                              heads.at[pl.ds(b, 1), :], sem).start()
        pltpu.make_async_copy(rel_hbm.at[pl.ds(ri, 1), :],
                              rels.at[pl.ds(b, 1), :], sem).start()

    s_ent = _PI / jnp.max(jnp.abs(entT_ref[...]))
    s_rel = _PI / jnp.max(jnp.abs(relT_ref[...]))
    scale_ref[0, 0] = s_ent

    for b in range(_B):
        pltpu.make_async_copy(ent_hbm.at[pl.ds(0, 1), :],
                              heads.at[pl.ds(b, 1), :], sem).wait()
        pltpu.make_async_copy(rel_hbm.at[pl.ds(0, 1), :],
                              rels.at[pl.ds(b, 1), :], sem).wait()

    a = heads[...] * s_ent + rels[...] * s_rel
    sinA_ref[...] = jnp.sin(a)
    cosA_ref[...] = jnp.cos(a)


def _sweep_body(scale_ref, sinA_ref, cosA_ref, entT_ref, out_ref):
    s = scale_ref[0, 0]
    p = entT_ref[...] * s
    sp, cp = _poly_sin_cos(p)
    dn = (((1,), (0,)), ((), ()))
    out_ref[...] = (
        jax.lax.dot_general(cosA_ref[...], sp, dn,
                            preferred_element_type=jnp.float32)
        - jax.lax.dot_general(sinA_ref[...], cp, dn,
                              preferred_element_type=jnp.float32)
    )


@jax.jit
def kernel(triples, ent_emb, rel_emb):
    num_ent, d = ent_emb.shape
    entT = ent_emb.T
    relT = rel_emb.T
    triples = triples.astype(jnp.int32)

    sinA, cosA, scale = pl.pallas_call(
        _prep_body,
        grid=(),
        in_specs=[
            pl.BlockSpec(memory_space=pltpu.SMEM),
            pl.BlockSpec(memory_space=pltpu.VMEM),
            pl.BlockSpec(memory_space=pltpu.VMEM),
            pl.BlockSpec(memory_space=pl.ANY),
            pl.BlockSpec(memory_space=pl.ANY),
        ],
        out_specs=[
            pl.BlockSpec(memory_space=pltpu.VMEM),
            pl.BlockSpec(memory_space=pltpu.VMEM),
            pl.BlockSpec(memory_space=pltpu.SMEM),
        ],
        out_shape=[
            jax.ShapeDtypeStruct((_B, _D), jnp.float32),
            jax.ShapeDtypeStruct((_B, _D), jnp.float32),
            jax.ShapeDtypeStruct((1, 1), jnp.float32),
        ],
        scratch_shapes=[
            pltpu.VMEM((_B, _D), jnp.float32),
            pltpu.VMEM((_B, _D), jnp.float32),
            pltpu.SemaphoreType.DMA,
        ],
    )(triples, entT, relT, ent_emb, rel_emb)

    bn = 50048
    grid = (num_ent + bn - 1) // bn
    out = pl.pallas_call(
        _sweep_body,
        grid=(grid,),
        in_specs=[
            pl.BlockSpec(memory_space=pltpu.SMEM),
            pl.BlockSpec((_B, _D), lambda i: (0, 0)),
            pl.BlockSpec((_B, _D), lambda i: (0, 0)),
            pl.BlockSpec((_D, bn), lambda i: (0, i)),
        ],
        out_specs=pl.BlockSpec((_B, bn), lambda i: (0, i)),
        out_shape=jax.ShapeDtypeStruct((_B, num_ent), jnp.float32),
    )(scale, sinA, cosA, entT)
    return out


# speedup vs baseline: 1.0061x; 1.0061x over previous
import jax
import jax.numpy as jnp
from jax.experimental import pallas as pl
from jax.experimental.pallas import tpu as pltpu

_PI = 3.141592653589793
_B = 32
_D = 16

_S = (0.9999994487625018, -0.1666652052026121, 0.008332213456463705,
      -0.0001980409467322523, 2.6949810250816164e-06,
      -2.0183356264080743e-08)
_C = (0.9999988662725678, -0.49999236445676765, 0.04165820791662088,
      -0.0013854161244634651, 2.4147561505205584e-05,
      -2.1742084399534178e-07)


def _poly_sin_cos(p):
    x2 = p * p
    sp = _S[5]
    for c in _S[4::-1]:
        sp = sp * x2 + c
    sp = sp * p
    cp = _C[5]
    for c in _C[4::-1]:
        cp = cp * x2 + c
    return sp, cp


def _prep_body(triples_ref, entT_ref, relT_ref, ent_hbm, rel_hbm,
               sinA_ref, cosA_ref, scale_ref, heads, rels, sem):
    for b in range(_B):
        hi = triples_ref[b, 0]
        ri = triples_ref[b, 1]
        pltpu.make_async_copy(ent_hbm.at[pl.ds(hi, 1), :],
                              heads.at[pl.ds(b, 1), :], sem).start()
        pltpu.make_async_copy(rel_hbm.at[pl.ds(ri, 1), :],
                              rels.at[pl.ds(b, 1), :], sem).start()

    s_ent = _PI / jnp.max(jnp.abs(entT_ref[...]))
    s_rel = _PI / jnp.max(jnp.abs(relT_ref[...]))
    scale_ref[0, 0] = s_ent

    for b in range(_B):
        pltpu.make_async_copy(ent_hbm.at[pl.ds(0, 1), :],
                              heads.at[pl.ds(b, 1), :], sem).wait()
        pltpu.make_async_copy(rel_hbm.at[pl.ds(0, 1), :],
                              rels.at[pl.ds(b, 1), :], sem).wait()

    a = heads[...] * s_ent + rels[...] * s_rel
    sinA_ref[...] = jnp.sin(a)
    cosA_ref[...] = jnp.cos(a)


def _sweep_body(scale_ref, sinA_ref, cosA_ref, entT_ref, out_ref):
    s = scale_ref[0, 0]
    p = entT_ref[...] * s
    sp, cp = _poly_sin_cos(p)
    dn = (((1,), (0,)), ((), ()))
    out_ref[...] = (
        jax.lax.dot_general(cosA_ref[...], sp, dn,
                            preferred_element_type=jnp.float32)
        - jax.lax.dot_general(sinA_ref[...], cp, dn,
                              preferred_element_type=jnp.float32)
    )


@jax.jit
def kernel(triples, ent_emb, rel_emb):
    num_ent, d = ent_emb.shape
    entT = ent_emb.T
    relT = rel_emb.T
    triples = triples.astype(jnp.int32)

    sinA, cosA, scale = pl.pallas_call(
        _prep_body,
        grid=(),
        in_specs=[
            pl.BlockSpec(memory_space=pltpu.SMEM),
            pl.BlockSpec(memory_space=pltpu.VMEM),
            pl.BlockSpec(memory_space=pltpu.VMEM),
            pl.BlockSpec(memory_space=pl.ANY),
            pl.BlockSpec(memory_space=pl.ANY),
        ],
        out_specs=[
            pl.BlockSpec(memory_space=pltpu.VMEM),
            pl.BlockSpec(memory_space=pltpu.VMEM),
            pl.BlockSpec(memory_space=pltpu.SMEM),
        ],
        out_shape=[
            jax.ShapeDtypeStruct((_B, _D), jnp.float32),
            jax.ShapeDtypeStruct((_B, _D), jnp.float32),
            jax.ShapeDtypeStruct((1, 1), jnp.float32),
        ],
        scratch_shapes=[
            pltpu.VMEM((_B, _D), jnp.float32),
            pltpu.VMEM((_B, _D), jnp.float32),
            pltpu.SemaphoreType.DMA,
        ],
    )(triples, entT, relT, ent_emb, rel_emb)

    bn = 25088
    grid = (num_ent + bn - 1) // bn
    out = pl.pallas_call(
        _sweep_body,
        grid=(grid,),
        in_specs=[
            pl.BlockSpec(memory_space=pltpu.SMEM),
            pl.BlockSpec((_B, _D), lambda i: (0, 0)),
            pl.BlockSpec((_B, _D), lambda i: (0, 0)),
            pl.BlockSpec((_D, bn), lambda i: (0, i)),
        ],
        out_specs=pl.BlockSpec((_B, bn), lambda i: (0, i)),
        out_shape=jax.ShapeDtypeStruct((_B, num_ent), jnp.float32),
    )(scale, sinA, cosA, entT)
    return out
